# baseline (speedup 1.0000x reference)
import functools

import jax
import jax.numpy as jnp
from jax import lax
from jax.experimental import pallas as pl
from jax.experimental.pallas import tpu as pltpu
from jax.experimental.pallas import tpu_sc as plsc

VOCAB = 1000000
D = 64
N = 4096 * 200

NC = 2
NS = 16
NW = NC * NS
ROWS_PER_W = N // NW
SUB = 128
CHUNK = 512
N_SUB = CHUNK // SUB
N_CHUNKS = ROWS_PER_W // CHUNK
RT = N // SUB
CT = D // 8

_LN2 = 0.6931471805599453

_sc_mesh = plsc.VectorSubcoreMesh(core_axis_name="c", subcore_axis_name="s")


def _ln(v):
    bits = plsc.bitcast(v, jnp.int32)
    e = ((bits >> 23) & 0xFF) - 127
    m = plsc.bitcast((bits & 0x007FFFFF) | 0x3F800000, jnp.float32)
    z = (m - 1.0) / (m + 1.0)
    z2 = z * z
    p = 1.0 / 7.0 + z2 * (1.0 / 9.0)
    p = 1.0 / 5.0 + z2 * p
    p = 1.0 / 3.0 + z2 * p
    lnm = 2.0 * z * (1.0 + z2 * p)
    return lnm + e.astype(jnp.float32) * _LN2


@functools.partial(
    pl.kernel,
    mesh=_sc_mesh,
    out_type=(
        jax.ShapeDtypeStruct((CT, RT // N_SUB, N_SUB * 8 * SUB), jnp.float32),
        jax.ShapeDtypeStruct((NW, 16), jnp.float32),
    ),
    scratch_types=[
        pltpu.VMEM((N_SUB, SUB), jnp.int32),
        pltpu.VMEM((CHUNK,), jnp.int32),
        pltpu.VMEM((CHUNK, D), jnp.float32),
        pltpu.VMEM((16,), jnp.float32),
        pltpu.VMEM((CHUNK * D,), jnp.float32),
        pltpu.SemaphoreType.DMA,
        pltpu.SemaphoreType.DMA,
    ],
    compiler_params=pltpu.CompilerParams(
        use_tc_tiling_on_sc=False, needs_layout_passes=False),
)
def _sc_embed_ce(idx_hbm, tgt_hbm, table_hbm, out_hbm, part_hbm,
                 idx_v, tgt_v, buf, accv, tbig, sem, sem_t):
    wid = lax.axis_index("s") * NC + lax.axis_index("c")
    grp0 = wid * (ROWS_PER_W // SUB)
    accv[...] = jnp.zeros((16,), jnp.float32)
    lane = lax.iota(jnp.int32, 16)

    def chunk_body(c, carry):
        g = grp0 + c * N_SUB
        pltpu.sync_copy(idx_hbm.at[pl.ds(g, N_SUB)], idx_v)
        pltpu.sync_copy(tgt_hbm.at[pl.ds(g * SUB, CHUNK)], tgt_v)
        gh = [
            pltpu.async_copy(
                table_hbm.at[idx_v.at[j]],
                buf.at[pl.ds(j * SUB, SUB)],
                sem,
            )
            for j in range(N_SUB)
        ]
        for h in gh:
            h.wait()

        def grp_body(gi, carry2):
            row0 = gi * 16
            rowvec = row0 + lane
            b = gi >> 3
            q0 = (gi & 7) * 16
            dyn = b * (8 * SUB) + q0
            s_vec = jnp.zeros((16,), jnp.float32)
            for col in range(D):
                v = plsc.load_gather(
                    buf, [rowvec, jnp.full((16,), col, jnp.int32)])
                s_vec = s_vec + jnp.exp(v)
                off = (col // 8) * (N_SUB * 8 * SUB) + (col % 8) * SUB
                tbig[pl.ds(dyn + off, 16)] = v
            tgt16 = tgt_v[pl.ds(row0, 16)]
            picked = plsc.load_gather(buf, [rowvec, tgt16])
            accv[...] = accv[...] + (_ln(s_vec) - picked)
            return carry2

        lax.fori_loop(0, CHUNK // 16, grp_body, 0)

        handles = []
        for ct in range(CT):
            handles.append(
                pltpu.async_copy(
                    tbig.at[pl.ds(ct * N_SUB * 8 * SUB, N_SUB * 8 * SUB)],
                    out_hbm.at[ct, wid * N_CHUNKS + c],
                    sem_t,
                )
            )
        for h in handles:
            h.wait()
        return carry

    lax.fori_loop(0, N_CHUNKS, chunk_body, 0)
    pltpu.sync_copy(accv, part_hbm.at[wid])


def kernel(inputs, targets, wte):
    idx2 = (inputs.astype(jnp.int32).reshape(-1) * 2).reshape(N // SUB, SUB)
    tgt = targets.astype(jnp.int32).reshape(N)
    table = jnp.pad(wte, ((0, 0), (0, 128 - D))).reshape(2 * VOCAB, D)
    out3d, partials = _sc_embed_ce(idx2, tgt, table)
    logits2 = (
        out3d.reshape(CT, RT, 8, SUB)
        .transpose(1, 3, 0, 2)
        .reshape(N, D)
    )
    loss = jnp.sum(partials) * (1.0 / N)
    return (logits2, loss)


# speedup vs baseline: 1.6355x; 1.6355x over previous
import functools

import jax
import jax.numpy as jnp
from jax import lax
from jax.experimental import pallas as pl
from jax.experimental.pallas import tpu as pltpu
from jax.experimental.pallas import tpu_sc as plsc

VOCAB = 1000000
D = 64
N = 4096 * 200

NC = 2
NS = 16
NW = NC * NS
ROWS_PER_W = N // NW
SUB = 128
CHUNK = 512
N_SUB = CHUNK // SUB
N_CHUNKS = ROWS_PER_W // CHUNK
N_PAIRS = N_CHUNKS // 2

_LN2 = 0.6931471805599453

_sc_mesh = plsc.VectorSubcoreMesh(core_axis_name="c", subcore_axis_name="s")


def _ln(v):
    bits = plsc.bitcast(v, jnp.int32)
    e = ((bits >> 23) & 0xFF) - 127
    m = plsc.bitcast((bits & 0x007FFFFF) | 0x3F800000, jnp.float32)
    z = (m - 1.0) / (m + 1.0)
    z2 = z * z
    p = 1.0 / 7.0 + z2 * (1.0 / 9.0)
    p = 1.0 / 5.0 + z2 * p
    p = 1.0 / 3.0 + z2 * p
    lnm = 2.0 * z * (1.0 + z2 * p)
    return lnm + e.astype(jnp.float32) * _LN2


@functools.partial(
    pl.kernel,
    mesh=_sc_mesh,
    out_type=(
        jax.ShapeDtypeStruct((N, D), jnp.float32),
        jax.ShapeDtypeStruct((NW, 16), jnp.float32),
    ),
    scratch_types=[
        [pltpu.VMEM((N_SUB, SUB), jnp.int32) for _ in range(2)],
        [pltpu.VMEM((CHUNK,), jnp.int32) for _ in range(2)],
        [pltpu.VMEM((CHUNK, D), jnp.float32) for _ in range(2)],
        pltpu.VMEM((16,), jnp.float32),
        [pltpu.SemaphoreType.DMA for _ in range(2)],
        [pltpu.SemaphoreType.DMA for _ in range(2)],
    ],
    compiler_params=pltpu.CompilerParams(
        use_tc_tiling_on_sc=False, needs_layout_passes=False),
)
def _sc_embed_ce(idx_hbm, tgt_hbm, table_hbm, out_hbm, part_hbm,
                 idx_v, tgt_v, bufs, accv, sems, semw):
    wid = lax.axis_index("s") * NC + lax.axis_index("c")
    grp0 = wid * (ROWS_PER_W // SUB)
    row0 = wid * ROWS_PER_W
    accv[...] = jnp.zeros((16,), jnp.float32)
    lane = lax.iota(jnp.int32, 16)

    def stage(c, slot):
        g = grp0 + c * N_SUB
        pltpu.sync_copy(idx_hbm.at[pl.ds(g, N_SUB)], idx_v[slot])
        pltpu.sync_copy(tgt_hbm.at[pl.ds(g * SUB, CHUNK)], tgt_v[slot])

    def fire(slot):
        return [
            pltpu.async_copy(
                table_hbm.at[idx_v[slot].at[j]],
                bufs[slot].at[pl.ds(j * SUB, SUB)],
                sems[slot],
            )
            for j in range(N_SUB)
        ]

    def ce(slot):
        buf = bufs[slot]
        tgt = tgt_v[slot]

        def grp_body(gi, carry2):
            r0 = gi * 16
            tgt16 = tgt[pl.ds(r0, 16)]
            s_vec = jnp.zeros((16,), jnp.float32)
            for r in range(16):
                e0 = jnp.exp(buf[r0 + r, pl.ds(0, 16)])
                e1 = jnp.exp(buf[r0 + r, pl.ds(16, 16)])
                e2 = jnp.exp(buf[r0 + r, pl.ds(32, 16)])
                e3 = jnp.exp(buf[r0 + r, pl.ds(48, 16)])
                s = jnp.sum((e0 + e1) + (e2 + e3))
                s_vec = jnp.where(lane == r, s, s_vec)
            picked = plsc.load_gather(buf, [r0 + lane, tgt16])
            accv[...] = accv[...] + (_ln(s_vec) - picked)
            return carry2

        lax.fori_loop(0, CHUNK // 16, grp_body, 0)

    def out_copy(c, slot):
        return pltpu.async_copy(
            bufs[slot],
            out_hbm.at[pl.ds(row0 + c * CHUNK, CHUNK)],
            semw[slot],
        )

    stage(0, 0)
    fire(0)

    def pair_body(p, carry):
        a = 2 * p
        stage(a + 1, 1)

        @pl.when(p > 0)
        def _():
            pltpu.make_async_copy(
                bufs[1],
                out_hbm.at[pl.ds(row0 + (a - 1) * CHUNK, CHUNK)],
                semw[1],
            ).wait()

        fire(1)
        for j in range(N_SUB):
            pltpu.make_async_copy(
                table_hbm.at[idx_v[0].at[j]],
                bufs[0].at[pl.ds(j * SUB, SUB)],
                sems[0],
            ).wait()
        ce(0)
        h0 = out_copy(a, 0)

        @pl.when(p < N_PAIRS - 1)
        def _():
            stage(a + 2, 0)
            h0.wait()
            fire(0)

        for j in range(N_SUB):
            pltpu.make_async_copy(
                table_hbm.at[idx_v[1].at[j]],
                bufs[1].at[pl.ds(j * SUB, SUB)],
                sems[1],
            ).wait()
        ce(1)
        out_copy(a + 1, 1)
        return carry

    lax.fori_loop(0, N_PAIRS, pair_body, 0)
    pltpu.make_async_copy(
        bufs[0],
        out_hbm.at[pl.ds(row0 + (N_CHUNKS - 2) * CHUNK, CHUNK)],
        semw[0],
    ).wait()
    pltpu.make_async_copy(
        bufs[1],
        out_hbm.at[pl.ds(row0 + (N_CHUNKS - 1) * CHUNK, CHUNK)],
        semw[1],
    ).wait()
    pltpu.sync_copy(accv, part_hbm.at[wid])


def kernel(inputs, targets, wte):
    idx2 = (inputs.astype(jnp.int32).reshape(-1) * 2).reshape(N // SUB, SUB)
    tgt = targets.astype(jnp.int32).reshape(N)
    table = jnp.pad(wte, ((0, 0), (0, 128 - D))).reshape(2 * VOCAB, D)
    logits2, partials = _sc_embed_ce(idx2, tgt, table)
    loss = jnp.sum(partials) * (1.0 / N)
    return (logits2, loss)
